# Initial kernel scaffold; baseline (speedup 1.0000x reference)
import jax
import jax.numpy as jnp
from jax.experimental import pallas as pl


def kernel(x, edge_index, edge_weight, params):
    raise NotImplementedError("write your pallas kernel here")


# speedup vs baseline: 2.5238x; 2.5238x over previous
import functools

import jax
import jax.numpy as jnp
from jax import lax
from jax.experimental import pallas as pl
from jax.experimental.pallas import tpu as pltpu
from jax.experimental.pallas import tpu_sc as plsc

N = 10000
E = 320000
SIGMA = 0.5
F32 = jnp.float32

BN = 1000
GRID = N // BN

NCORE = 2
NSUB = 16
BLK = 128
EPAD = 327680
NBLOCKS = EPAD // BLK
TBW = NBLOCKS // NSUB
TBS = NBLOCKS // (NSUB * NCORE)
SG = 16
ZR = 64
ROWS_A = 632
ROWS_LAST = N - (NSUB - 1) * ROWS_A


def _full(shape):
    return pl.BlockSpec(shape, lambda i: tuple(0 for _ in shape))


def _ae_body(x_ref, w1, b1, w2, b2, w3, b3, wz, bz, wd1, bd1, wd2, bd2,
             wd3, bd3, wx, bx, g1, cl,
             t1o, t2o, t3o, zo, xo, qo, s1o):
    x = x_ref[...]
    t1 = jnp.maximum(x @ w1[...] + b1[...], 0.0)
    t1o[...] = t1
    t2 = jnp.maximum(t1 @ w2[...] + b2[...], 0.0)
    t2o[...] = t2
    t3 = jnp.maximum(t2 @ w3[...] + b3[...], 0.0)
    t3o[...] = t3
    z = t3 @ wz[...] + bz[...]
    zo[...] = z
    d1 = jnp.maximum(z @ wd1[...] + bd1[...], 0.0)
    d2 = jnp.maximum(d1 @ wd2[...] + bd2[...], 0.0)
    d3 = jnp.maximum(d2 @ wd3[...] + bd3[...], 0.0)
    xo[...] = d3 @ wx[...] + bx[...]
    c = cl[...]
    d2q = (jnp.sum(z * z, axis=1, keepdims=True)
           - 2.0 * (z @ c.T)
           + jnp.sum(c * c, axis=1)[None, :])
    q = 1.0 / (1.0 + d2q)
    col = lax.broadcasted_iota(jnp.int32, q.shape, 1)
    q = jnp.where(col < 10, q, 0.0)
    qo[...] = q / jnp.sum(q, axis=1, keepdims=True)
    s1 = x @ g1[...]
    s1o[0] = s1[:, :128]
    s1o[1] = s1[:, 128:]


def _call_ae(x, p, cl_pad):
    outs = (
        jax.ShapeDtypeStruct((N, 256), F32),
        jax.ShapeDtypeStruct((N, 256), F32),
        jax.ShapeDtypeStruct((N, 512), F32),
        jax.ShapeDtypeStruct((N, 16), F32),
        jax.ShapeDtypeStruct((N, 128), F32),
        jax.ShapeDtypeStruct((N, 16), F32),
        jax.ShapeDtypeStruct((2, N, 128), F32),
    )
    args = [x,
            p['enc1_w'], p['enc1_b'][None, :], p['enc2_w'], p['enc2_b'][None, :],
            p['enc3_w'], p['enc3_b'][None, :], p['zlay_w'], p['zlay_b'][None, :],
            p['dec1_w'], p['dec1_b'][None, :], p['dec2_w'], p['dec2_b'][None, :],
            p['dec3_w'], p['dec3_b'][None, :], p['xbar_w'], p['xbar_b'][None, :],
            p['gnn1_w'], cl_pad]
    in_specs = [pl.BlockSpec((BN, 128), lambda i: (i, 0))]
    in_specs += [_full(a.shape) for a in args[1:]]
    out_specs = (
        pl.BlockSpec((BN, 256), lambda i: (i, 0)),
        pl.BlockSpec((BN, 256), lambda i: (i, 0)),
        pl.BlockSpec((BN, 512), lambda i: (i, 0)),
        pl.BlockSpec((BN, 16), lambda i: (i, 0)),
        pl.BlockSpec((BN, 128), lambda i: (i, 0)),
        pl.BlockSpec((BN, 16), lambda i: (i, 0)),
        pl.BlockSpec((2, BN, 128), lambda i: (0, i, 0)),
    )
    return pl.pallas_call(
        _ae_body, grid=(GRID,), in_specs=in_specs, out_specs=out_specs,
        out_shape=outs)(*args)


def _make_blend_wide(cout):
    def body(acc_ref, tra_ref, w_ref, out_ref):
        h = jnp.concatenate(
            [jnp.maximum(acc_ref[i], 0.0) for i in range(2)], axis=1)
        m = (1.0 - SIGMA) * h + SIGMA * tra_ref[...]
        s = m @ w_ref[...]
        for c in range(cout):
            out_ref[c] = s[:, c * 128:(c + 1) * 128]
    return body


def _call_blend_wide(acc, tra, w, cout):
    din = tra.shape[1]
    return pl.pallas_call(
        _make_blend_wide(cout), grid=(GRID,),
        in_specs=[pl.BlockSpec((2, BN, 128), lambda i: (0, i, 0)),
                  pl.BlockSpec((BN, din), lambda i: (i, 0)),
                  _full(w.shape)],
        out_specs=pl.BlockSpec((cout, BN, 128), lambda i: (0, i, 0)),
        out_shape=jax.ShapeDtypeStruct((cout, N, 128), F32))(acc, tra, w)


def _b3_body(acc_ref, tra_ref, w_ref, out_ref):
    h = jnp.concatenate(
        [jnp.maximum(acc_ref[i], 0.0) for i in range(4)], axis=1)
    m = (1.0 - SIGMA) * h + SIGMA * tra_ref[...]
    out_ref[...] = m @ w_ref[...]


def _call_b3(acc, tra, w_pad):
    return pl.pallas_call(
        _b3_body, grid=(GRID,),
        in_specs=[pl.BlockSpec((4, BN, 128), lambda i: (0, i, 0)),
                  pl.BlockSpec((BN, 512), lambda i: (i, 0)),
                  _full(w_pad.shape)],
        out_specs=pl.BlockSpec((BN, 128), lambda i: (i, 0)),
        out_shape=jax.ShapeDtypeStruct((N, 128), F32))(acc, tra, w_pad)


def _b4_body(acc_ref, z_ref, w_ref, out_ref):
    h = jnp.maximum(acc_ref[0][:, :16] + acc_ref[1][:, :16], 0.0)
    m = (1.0 - SIGMA) * h + SIGMA * z_ref[...]
    out_ref[...] = m @ w_ref[...]


def _call_b4(acc_pair, z, w_pad):
    return pl.pallas_call(
        _b4_body, grid=(GRID,),
        in_specs=[pl.BlockSpec((2, BN, 128), lambda i: (0, i, 0)),
                  pl.BlockSpec((BN, 16), lambda i: (i, 0)),
                  _full(w_pad.shape)],
        out_specs=pl.BlockSpec((BN, 128), lambda i: (i, 0)),
        out_shape=jax.ShapeDtypeStruct((N, 128), F32))(acc_pair, z, w_pad)


def _fin_body(acc_ref, out_ref):
    h = acc_ref[0][:, :16] + acc_ref[1][:, :16]
    col = lax.broadcasted_iota(jnp.int32, h.shape, 1)
    h = jnp.where(col < 10, h, -1e30)
    e = jnp.exp(h - jnp.max(h, axis=1, keepdims=True))
    out_ref[...] = e / jnp.sum(e, axis=1, keepdims=True)


def _call_fin(acc_pair):
    return pl.pallas_call(
        _fin_body, grid=(GRID,),
        in_specs=[pl.BlockSpec((2, BN, 128), lambda i: (0, i, 0))],
        out_specs=pl.BlockSpec((BN, 16), lambda i: (i, 0)),
        out_shape=jax.ShapeDtypeStruct((N, 16), F32))(acc_pair)


def _mesh():
    return plsc.VectorSubcoreMesh(core_axis_name="c", subcore_axis_name="s",
                                  num_cores=NCORE, num_subcores=NSUB)


def _zero_slice(accum, zbuf, r0, nr):
    full = nr // ZR
    for k in range(full):
        pltpu.sync_copy(zbuf, accum.at[pl.ds(r0 + k * ZR, ZR)])
    rem = nr - full * ZR
    if rem:
        pltpu.sync_copy(zbuf.at[pl.ds(0, rem)],
                        accum.at[pl.ds(r0 + full * ZR, rem)])


def _make_spmm_wide(n_chunks):
    @functools.partial(
        pl.kernel,
        out_type=jax.ShapeDtypeStruct((n_chunks, N, 128), F32),
        mesh=_mesh(),
        scratch_types=[
            pltpu.VMEM((SG, BLK), jnp.int32),
            pltpu.VMEM((SG, BLK), jnp.int32),
            pltpu.VMEM((SG, BLK), F32),
            pltpu.VMEM((BLK, 128), F32),
            pltpu.VMEM((ZR, 128), F32),
            pltpu.VMEM_SHARED((N, 128), F32),
            pltpu.SemaphoreType.DMA,
        ])
    def spmm(tab, cols_c, rows, w, out, cols_v, rows_v, w_v, rowbuf, zbuf,
             accum, sem):
        cid = lax.axis_index("c")
        sid = lax.axis_index("s")

        @pl.loop(0, ZR)
        def _zero_zbuf(i):
            for f in range(8):
                zbuf[i, pl.ds(f * 16, 16)] = jnp.zeros((16,), F32)

        r0 = sid * ROWS_A
        for ci in range(n_chunks // NCORE):
            chunk = cid + NCORE * ci

            @pl.when(sid < NSUB - 1)
            def _za():
                _zero_slice(accum, zbuf, r0, ROWS_A)

            @pl.when(sid == NSUB - 1)
            def _zl():
                _zero_slice(accum, zbuf, r0, ROWS_LAST)

            plsc.subcore_barrier()

            @pl.loop(0, TBW // SG)
            def _supergroup(s):
                o = pl.multiple_of(s * SG, SG)
                pltpu.sync_copy(cols_c.at[chunk, sid, pl.ds(o, SG)], cols_v)
                pltpu.sync_copy(rows.at[sid, pl.ds(o, SG)], rows_v)
                pltpu.sync_copy(w.at[sid, pl.ds(o, SG)], w_v)

                @pl.loop(0, SG)
                def _edge_block(j):
                    pltpu.async_copy(tab.at[cols_v.at[j]], rowbuf, sem).wait()

                    @pl.loop(0, BLK // 16)
                    def _edge_group(g):
                        wvec = w_v[j, pl.ds(g * 16, 16)]
                        for k in range(16):
                            e = g * 16 + k
                            wv = wvec[k]
                            for f in range(8):
                                sl = pl.ds(f * 16, 16)
                                rowbuf[e, sl] = rowbuf[e, sl] * wv

                    pltpu.sync_copy(rowbuf, accum.at[rows_v.at[j]], add=True)

            plsc.subcore_barrier()

            @pl.when(sid < NSUB - 1)
            def _da():
                pltpu.sync_copy(accum.at[pl.ds(r0, ROWS_A)],
                                out.at[chunk, pl.ds(r0, ROWS_A)])

            @pl.when(sid == NSUB - 1)
            def _dl():
                pltpu.sync_copy(accum.at[pl.ds(r0, ROWS_LAST)],
                                out.at[chunk, pl.ds(r0, ROWS_LAST)])

    return spmm


def _make_spmm_split():
    @functools.partial(
        pl.kernel,
        out_type=jax.ShapeDtypeStruct((NCORE, N, 128), F32),
        mesh=_mesh(),
        scratch_types=[
            pltpu.VMEM((SG, BLK), jnp.int32),
            pltpu.VMEM((SG, BLK), jnp.int32),
            pltpu.VMEM((SG, BLK), F32),
            pltpu.VMEM((BLK, 128), F32),
            pltpu.VMEM((ZR, 128), F32),
            pltpu.VMEM_SHARED((N, 128), F32),
            pltpu.SemaphoreType.DMA,
        ])
    def spmm(tab, cols, rows, w, out, cols_v, rows_v, w_v, rowbuf, zbuf,
             accum, sem):
        cid = lax.axis_index("c")
        sid = lax.axis_index("s")
        wid = cid * NSUB + sid

        @pl.loop(0, ZR)
        def _zero_zbuf(i):
            for f in range(8):
                zbuf[i, pl.ds(f * 16, 16)] = jnp.zeros((16,), F32)

        r0 = sid * ROWS_A

        @pl.when(sid < NSUB - 1)
        def _za():
            _zero_slice(accum, zbuf, r0, ROWS_A)

        @pl.when(sid == NSUB - 1)
        def _zl():
            _zero_slice(accum, zbuf, r0, ROWS_LAST)

        plsc.subcore_barrier()

        @pl.loop(0, TBS // SG)
        def _supergroup(s):
            o = pl.multiple_of(s * SG, SG)
            pltpu.sync_copy(cols.at[wid, pl.ds(o, SG)], cols_v)
            pltpu.sync_copy(rows.at[wid, pl.ds(o, SG)], rows_v)
            pltpu.sync_copy(w.at[wid, pl.ds(o, SG)], w_v)

            @pl.loop(0, SG)
            def _edge_block(j):
                pltpu.async_copy(tab.at[cols_v.at[j]], rowbuf, sem).wait()

                @pl.loop(0, BLK // 16)
                def _edge_group(g):
                    wvec = w_v[j, pl.ds(g * 16, 16)]
                    for k in range(16):
                        e = g * 16 + k
                        wv = wvec[k]
                        for f in range(8):
                            sl = pl.ds(f * 16, 16)
                            rowbuf[e, sl] = rowbuf[e, sl] * wv

                pltpu.sync_copy(rowbuf, accum.at[rows_v.at[j]], add=True)

        plsc.subcore_barrier()

        @pl.when(sid < NSUB - 1)
        def _da():
            pltpu.sync_copy(accum.at[pl.ds(r0, ROWS_A)],
                            out.at[cid, pl.ds(r0, ROWS_A)])

        @pl.when(sid == NSUB - 1)
        def _dl():
            pltpu.sync_copy(accum.at[pl.ds(r0, ROWS_LAST)],
                            out.at[cid, pl.ds(r0, ROWS_LAST)])

    return spmm


@functools.cache
def _spmm_wide(n_chunks):
    return _make_spmm_wide(n_chunks)


@functools.cache
def _spmm_split():
    return _make_spmm_split()


def kernel(x, edge_index, edge_weight, params):
    p = params
    rows = edge_index[0]
    cols = edge_index[1]
    pad = EPAD - E
    rows_p = jnp.pad(rows, (0, pad))
    cols_p = jnp.pad(cols, (0, pad))
    w_p = jnp.pad(edge_weight, (0, pad))
    rows_w = rows_p.reshape(NSUB, TBW, BLK)
    cols_w = cols_p.reshape(NSUB, TBW, BLK)
    ww = w_p.reshape(NSUB, TBW, BLK)
    cols_c2 = jnp.stack([cols_w, cols_w + N])
    cols_c4 = jnp.stack([cols_w + c * N for c in range(4)])
    rows_n = rows_p.reshape(NSUB * NCORE, TBS, BLK)
    cols_n = cols_p.reshape(NSUB * NCORE, TBS, BLK)
    wn = w_p.reshape(NSUB * NCORE, TBS, BLK)

    cl_pad = jnp.zeros((16, 16), F32).at[:10].set(p['cluster'])
    g4_pad = jnp.zeros((512, 128), F32).at[:, :16].set(p['gnn4_w'])
    g5_pad = jnp.zeros((16, 128), F32).at[:, :10].set(p['gnn5_w'])

    tra1, tra2, tra3, z, x_bar, q_pad, s1c = _call_ae(x, p, cl_pad)

    acc1 = _spmm_wide(2)(s1c.reshape(2 * N, 128), cols_c2, rows_w, ww)
    s2c = _call_blend_wide(acc1, tra1, p['gnn2_w'], 2)
    acc2 = _spmm_wide(2)(s2c.reshape(2 * N, 128), cols_c2, rows_w, ww)
    s3c = _call_blend_wide(acc2, tra2, p['gnn3_w'], 4)
    acc3 = _spmm_wide(4)(s3c.reshape(4 * N, 128), cols_c4, rows_w, ww)
    s4 = _call_b3(acc3, tra3, g4_pad)
    acc4 = _spmm_split()(s4, cols_n, rows_n, wn)
    s5 = _call_b4(acc4, z, g5_pad)
    acc5 = _spmm_split()(s5, cols_n, rows_n, wn)
    pred_pad = _call_fin(acc5)

    return (x_bar, q_pad[:, :10], pred_pad[:, :10], z)


# speedup vs baseline: 2.8295x; 1.1211x over previous
import functools

import jax
import jax.numpy as jnp
from jax import lax
from jax.experimental import pallas as pl
from jax.experimental.pallas import tpu as pltpu
from jax.experimental.pallas import tpu_sc as plsc

N = 10000
E = 320000
SIGMA = 0.5
F32 = jnp.float32

BN = 1000
GRID = N // BN

NCORE = 2
NSUB = 16
BLK = 64
EPAD = 327680
TBW = EPAD // NSUB // BLK
TBS = EPAD // (NSUB * NCORE) // BLK
SG = 16
NSGW = TBW // SG
NSGS = TBS // SG
NBUF = 4
ROWS_A = 632
ROWS_LAST = N - (NSUB - 1) * ROWS_A


def _full(shape):
    return pl.BlockSpec(shape, lambda i: tuple(0 for _ in shape))


def _ae_body(x_ref, w1, b1, w2, b2, w3, b3, wz, bz, wd1, bd1, wd2, bd2,
             wd3, bd3, wx, bx, g1, cl,
             t1o, t2o, t3o, zo, xo, qo, s1o):
    x = x_ref[...]
    t1 = jnp.maximum(x @ w1[...] + b1[...], 0.0)
    t1o[...] = t1
    t2 = jnp.maximum(t1 @ w2[...] + b2[...], 0.0)
    t2o[...] = t2
    t3 = jnp.maximum(t2 @ w3[...] + b3[...], 0.0)
    t3o[...] = t3
    z = t3 @ wz[...] + bz[...]
    zo[...] = z
    d1 = jnp.maximum(z @ wd1[...] + bd1[...], 0.0)
    d2 = jnp.maximum(d1 @ wd2[...] + bd2[...], 0.0)
    d3 = jnp.maximum(d2 @ wd3[...] + bd3[...], 0.0)
    xo[...] = d3 @ wx[...] + bx[...]
    c = cl[...]
    d2q = (jnp.sum(z * z, axis=1, keepdims=True)
           - 2.0 * (z @ c.T)
           + jnp.sum(c * c, axis=1)[None, :])
    q = 1.0 / (1.0 + d2q)
    col = lax.broadcasted_iota(jnp.int32, q.shape, 1)
    q = jnp.where(col < 10, q, 0.0)
    qo[...] = q / jnp.sum(q, axis=1, keepdims=True)
    s1 = x @ g1[...]
    s1o[0] = s1[:, :128]
    s1o[1] = s1[:, 128:]


def _call_ae(x, p, cl_pad):
    outs = (
        jax.ShapeDtypeStruct((N, 256), F32),
        jax.ShapeDtypeStruct((N, 256), F32),
        jax.ShapeDtypeStruct((N, 512), F32),
        jax.ShapeDtypeStruct((N, 16), F32),
        jax.ShapeDtypeStruct((N, 128), F32),
        jax.ShapeDtypeStruct((N, 16), F32),
        jax.ShapeDtypeStruct((2, N, 128), F32),
    )
    args = [x,
            p['enc1_w'], p['enc1_b'][None, :], p['enc2_w'], p['enc2_b'][None, :],
            p['enc3_w'], p['enc3_b'][None, :], p['zlay_w'], p['zlay_b'][None, :],
            p['dec1_w'], p['dec1_b'][None, :], p['dec2_w'], p['dec2_b'][None, :],
            p['dec3_w'], p['dec3_b'][None, :], p['xbar_w'], p['xbar_b'][None, :],
            p['gnn1_w'], cl_pad]
    in_specs = [pl.BlockSpec((BN, 128), lambda i: (i, 0))]
    in_specs += [_full(a.shape) for a in args[1:]]
    out_specs = (
        pl.BlockSpec((BN, 256), lambda i: (i, 0)),
        pl.BlockSpec((BN, 256), lambda i: (i, 0)),
        pl.BlockSpec((BN, 512), lambda i: (i, 0)),
        pl.BlockSpec((BN, 16), lambda i: (i, 0)),
        pl.BlockSpec((BN, 128), lambda i: (i, 0)),
        pl.BlockSpec((BN, 16), lambda i: (i, 0)),
        pl.BlockSpec((2, BN, 128), lambda i: (0, i, 0)),
    )
    return pl.pallas_call(
        _ae_body, grid=(GRID,), in_specs=in_specs, out_specs=out_specs,
        out_shape=outs)(*args)


def _make_blend_wide(cout):
    def body(acc_ref, tra_ref, w_ref, out_ref):
        h = jnp.concatenate(
            [jnp.maximum(acc_ref[i], 0.0) for i in range(2)], axis=1)
        m = (1.0 - SIGMA) * h + SIGMA * tra_ref[...]
        s = m @ w_ref[...]
        for c in range(cout):
            out_ref[c] = s[:, c * 128:(c + 1) * 128]
    return body


def _call_blend_wide(acc, tra, w, cout):
    din = tra.shape[1]
    return pl.pallas_call(
        _make_blend_wide(cout), grid=(GRID,),
        in_specs=[pl.BlockSpec((2, BN, 128), lambda i: (0, i, 0)),
                  pl.BlockSpec((BN, din), lambda i: (i, 0)),
                  _full(w.shape)],
        out_specs=pl.BlockSpec((cout, BN, 128), lambda i: (0, i, 0)),
        out_shape=jax.ShapeDtypeStruct((cout, N, 128), F32))(acc, tra, w)


def _b3_body(acc_ref, tra_ref, w_ref, out_ref):
    h = jnp.concatenate(
        [jnp.maximum(acc_ref[i], 0.0) for i in range(4)], axis=1)
    m = (1.0 - SIGMA) * h + SIGMA * tra_ref[...]
    out_ref[...] = m @ w_ref[...]


def _call_b3(acc, tra, w_pad):
    return pl.pallas_call(
        _b3_body, grid=(GRID,),
        in_specs=[pl.BlockSpec((4, BN, 128), lambda i: (0, i, 0)),
                  pl.BlockSpec((BN, 512), lambda i: (i, 0)),
                  _full(w_pad.shape)],
        out_specs=pl.BlockSpec((BN, 128), lambda i: (i, 0)),
        out_shape=jax.ShapeDtypeStruct((N, 128), F32))(acc, tra, w_pad)


def _b4_body(acc_ref, z_ref, w_ref, out_ref):
    h = jnp.maximum(acc_ref[0][:, :16] + acc_ref[1][:, :16], 0.0)
    m = (1.0 - SIGMA) * h + SIGMA * z_ref[...]
    out_ref[...] = m @ w_ref[...]


def _call_b4(acc_pair, z, w_pad):
    return pl.pallas_call(
        _b4_body, grid=(GRID,),
        in_specs=[pl.BlockSpec((2, BN, 128), lambda i: (0, i, 0)),
                  pl.BlockSpec((BN, 16), lambda i: (i, 0)),
                  _full(w_pad.shape)],
        out_specs=pl.BlockSpec((BN, 128), lambda i: (i, 0)),
        out_shape=jax.ShapeDtypeStruct((N, 128), F32))(acc_pair, z, w_pad)


def _fin_body(acc_ref, out_ref):
    h = acc_ref[0][:, :16] + acc_ref[1][:, :16]
    col = lax.broadcasted_iota(jnp.int32, h.shape, 1)
    h = jnp.where(col < 10, h, -1e30)
    e = jnp.exp(h - jnp.max(h, axis=1, keepdims=True))
    out_ref[...] = e / jnp.sum(e, axis=1, keepdims=True)


def _call_fin(acc_pair):
    return pl.pallas_call(
        _fin_body, grid=(GRID,),
        in_specs=[pl.BlockSpec((2, BN, 128), lambda i: (0, i, 0))],
        out_specs=pl.BlockSpec((BN, 16), lambda i: (i, 0)),
        out_shape=jax.ShapeDtypeStruct((N, 16), F32))(acc_pair)


def _mesh():
    return plsc.VectorSubcoreMesh(core_axis_name="c", subcore_axis_name="s",
                                  num_cores=NCORE, num_subcores=NSUB)


def _zero_slice(accum, zbuf, r0, nr):
    full = nr // BLK
    for k in range(full):
        pltpu.sync_copy(zbuf, accum.at[pl.ds(r0 + k * BLK, BLK)])
    rem = nr - full * BLK
    if rem:
        pltpu.sync_copy(zbuf.at[pl.ds(0, rem)],
                        accum.at[pl.ds(r0 + full * BLK, rem)])


def _zero_phase(accum, zbuf, sid):
    @pl.loop(0, BLK)
    def _zb(i):
        for f in range(8):
            zbuf[i, pl.ds(f * 16, 16)] = jnp.zeros((16,), F32)

    r0 = sid * ROWS_A

    @pl.when(sid < NSUB - 1)
    def _za():
        _zero_slice(accum, zbuf, r0, ROWS_A)

    @pl.when(sid == NSUB - 1)
    def _zl():
        _zero_slice(accum, zbuf, r0, ROWS_LAST)


def _dump_phase(accum, out_view, sid):
    r0 = sid * ROWS_A

    @pl.when(sid < NSUB - 1)
    def _da():
        pltpu.sync_copy(accum.at[pl.ds(r0, ROWS_A)],
                        out_view.at[pl.ds(r0, ROWS_A)])

    @pl.when(sid == NSUB - 1)
    def _dl():
        pltpu.sync_copy(accum.at[pl.ds(r0, ROWS_LAST)],
                        out_view.at[pl.ds(r0, ROWS_LAST)])


def _edge_sweep(tab, accum, bufs, gsems, ssems, cols_v, rows_v, w_v,
                stage, nsg):
    @pl.loop(0, nsg)
    def _sg(s):
        p = s % 2
        stage(s, p)
        pltpu.async_copy(tab.at[cols_v.at[p, 0]], bufs[0], gsems[0])

        @pl.loop(0, SG // NBUF)
        def _quad(u):
            for t in range(NBUF):
                l = u * NBUF + t
                jg = s * SG + l
                P = bufs[t]
                tn = (t + 1) % NBUF

                pltpu.make_async_copy(tab.at[cols_v.at[p, l]], P,
                                      gsems[t]).wait()

                @pl.when(jg >= NBUF - 1)
                def _ws():
                    pltpu.make_async_copy(
                        bufs[tn], accum.at[rows_v.at[p, l]],
                        ssems[tn]).wait()

                if t < NBUF - 1:
                    pltpu.async_copy(tab.at[cols_v.at[p, l + 1]],
                                     bufs[tn], gsems[tn])
                else:
                    @pl.when(u < SG // NBUF - 1)
                    def _gn():
                        pltpu.async_copy(tab.at[cols_v.at[p, l + 1]],
                                         bufs[tn], gsems[tn])

                @pl.loop(0, BLK // 16)
                def _mult(g):
                    wvec = w_v[p, l, pl.ds(g * 16, 16)]
                    for k in range(16):
                        wv = wvec[k]
                        e = g * 16 + k
                        for f in range(8):
                            sl = pl.ds(f * 16, 16)
                            P[e, sl] = P[e, sl] * wv

                pltpu.async_copy(P, accum.at[rows_v.at[p, l]], ssems[t],
                                 add=True)

    for t in range(1, NBUF):
        pltpu.make_async_copy(bufs[t], accum.at[rows_v.at[0, 0]],
                              ssems[t]).wait()


def _make_spmm_wide(n_chunks):
    @functools.partial(
        pl.kernel,
        out_type=jax.ShapeDtypeStruct((n_chunks, N, 128), F32),
        mesh=_mesh(),
        scratch_types=[
            pltpu.VMEM((2, SG, BLK), jnp.int32),
            pltpu.VMEM((2, SG, BLK), jnp.int32),
            pltpu.VMEM((2, SG, BLK), F32),
            pltpu.VMEM((BLK, 128), F32),
            pltpu.VMEM((BLK, 128), F32),
            pltpu.VMEM((BLK, 128), F32),
            pltpu.VMEM((BLK, 128), F32),
            pltpu.VMEM_SHARED((N, 128), F32),
            pltpu.SemaphoreType.DMA,
            pltpu.SemaphoreType.DMA,
            pltpu.SemaphoreType.DMA,
            pltpu.SemaphoreType.DMA,
            pltpu.SemaphoreType.DMA,
            pltpu.SemaphoreType.DMA,
            pltpu.SemaphoreType.DMA,
            pltpu.SemaphoreType.DMA,
        ])
    def spmm(tab, cols_c, rows, w, out, cols_v, rows_v, w_v,
             b0, b1, b2, b3, accum,
             g0, g1, g2, g3, s0, s1, s2, s3):
        cid = lax.axis_index("c")
        sid = lax.axis_index("s")
        bufs = (b0, b1, b2, b3)
        gsems = (g0, g1, g2, g3)
        ssems = (s0, s1, s2, s3)

        for ci in range(n_chunks // NCORE):
            chunk = cid + NCORE * ci
            _zero_phase(accum, b0, sid)
            plsc.subcore_barrier()

            def stage(s, p, _chunk=chunk):
                o = pl.multiple_of(s * SG, SG)
                pltpu.sync_copy(cols_c.at[_chunk, sid, pl.ds(o, SG)],
                                cols_v.at[p])
                pltpu.sync_copy(rows.at[sid, pl.ds(o, SG)], rows_v.at[p])
                pltpu.sync_copy(w.at[sid, pl.ds(o, SG)], w_v.at[p])

            _edge_sweep(tab, accum, bufs, gsems, ssems, cols_v, rows_v, w_v,
                        stage, NSGW)
            plsc.subcore_barrier()
            _dump_phase(accum, out.at[chunk], sid)

    return spmm


def _make_spmm_split():
    @functools.partial(
        pl.kernel,
        out_type=jax.ShapeDtypeStruct((NCORE, N, 128), F32),
        mesh=_mesh(),
        scratch_types=[
            pltpu.VMEM((2, SG, BLK), jnp.int32),
            pltpu.VMEM((2, SG, BLK), jnp.int32),
            pltpu.VMEM((2, SG, BLK), F32),
            pltpu.VMEM((BLK, 128), F32),
            pltpu.VMEM((BLK, 128), F32),
            pltpu.VMEM((BLK, 128), F32),
            pltpu.VMEM((BLK, 128), F32),
            pltpu.VMEM_SHARED((N, 128), F32),
            pltpu.SemaphoreType.DMA,
            pltpu.SemaphoreType.DMA,
            pltpu.SemaphoreType.DMA,
            pltpu.SemaphoreType.DMA,
            pltpu.SemaphoreType.DMA,
            pltpu.SemaphoreType.DMA,
            pltpu.SemaphoreType.DMA,
            pltpu.SemaphoreType.DMA,
        ])
    def spmm(tab, cols, rows, w, out, cols_v, rows_v, w_v,
             b0, b1, b2, b3, accum,
             g0, g1, g2, g3, s0, s1, s2, s3):
        cid = lax.axis_index("c")
        sid = lax.axis_index("s")
        wid = cid * NSUB + sid
        bufs = (b0, b1, b2, b3)
        gsems = (g0, g1, g2, g3)
        ssems = (s0, s1, s2, s3)

        _zero_phase(accum, b0, sid)
        plsc.subcore_barrier()

        def stage(s, p):
            o = pl.multiple_of(s * SG, SG)
            pltpu.sync_copy(cols.at[wid, pl.ds(o, SG)], cols_v.at[p])
            pltpu.sync_copy(rows.at[wid, pl.ds(o, SG)], rows_v.at[p])
            pltpu.sync_copy(w.at[wid, pl.ds(o, SG)], w_v.at[p])

        _edge_sweep(tab, accum, bufs, gsems, ssems, cols_v, rows_v, w_v,
                    stage, NSGS)
        plsc.subcore_barrier()
        _dump_phase(accum, out.at[cid], sid)

    return spmm


@functools.cache
def _spmm_wide(n_chunks):
    return _make_spmm_wide(n_chunks)


@functools.cache
def _spmm_split():
    return _make_spmm_split()


def kernel(x, edge_index, edge_weight, params):
    p = params
    rows = edge_index[0]
    cols = edge_index[1]
    pad = EPAD - E
    rows_p = jnp.pad(rows, (0, pad))
    cols_p = jnp.pad(cols, (0, pad))
    w_p = jnp.pad(edge_weight, (0, pad))
    rows_w = rows_p.reshape(NSUB, TBW, BLK)
    cols_w = cols_p.reshape(NSUB, TBW, BLK)
    ww = w_p.reshape(NSUB, TBW, BLK)
    cols_c2 = jnp.stack([cols_w, cols_w + N])
    cols_c4 = jnp.stack([cols_w + c * N for c in range(4)])
    rows_n = rows_p.reshape(NSUB * NCORE, TBS, BLK)
    cols_n = cols_p.reshape(NSUB * NCORE, TBS, BLK)
    wn = w_p.reshape(NSUB * NCORE, TBS, BLK)

    cl_pad = jnp.zeros((16, 16), F32).at[:10].set(p['cluster'])
    g4_pad = jnp.zeros((512, 128), F32).at[:, :16].set(p['gnn4_w'])
    g5_pad = jnp.zeros((16, 128), F32).at[:, :10].set(p['gnn5_w'])

    tra1, tra2, tra3, z, x_bar, q_pad, s1c = _call_ae(x, p, cl_pad)

    acc1 = _spmm_wide(2)(s1c.reshape(2 * N, 128), cols_c2, rows_w, ww)
    s2c = _call_blend_wide(acc1, tra1, p['gnn2_w'], 2)
    acc2 = _spmm_wide(2)(s2c.reshape(2 * N, 128), cols_c2, rows_w, ww)
    s3c = _call_blend_wide(acc2, tra2, p['gnn3_w'], 4)
    acc3 = _spmm_wide(4)(s3c.reshape(4 * N, 128), cols_c4, rows_w, ww)
    s4 = _call_b3(acc3, tra3, g4_pad)
    acc4 = _spmm_split()(s4, cols_n, rows_n, wn)
    s5 = _call_b4(acc4, z, g5_pad)
    acc5 = _spmm_split()(s5, cols_n, rows_n, wn)
    pred_pad = _call_fin(acc5)

    return (x_bar, q_pad[:, :10], pred_pad[:, :10], z)
